# baseline (speedup 1.0000x reference)
import functools

import jax
import jax.numpy as jnp
from jax.experimental import pallas as pl
from jax.experimental.pallas import tpu as pltpu

_K = 3
_RB = 4


def _se_kernel(x_hbm, w1t_ref, b1_ref, w2t_ref, b2_ref, o_hbm,
               xbuf, obuf, insem, outsem, *, inv_t, nt_per_core):
    core = pl.program_id(0)
    base = core * nt_per_core

    def dma_in(slot, i):
        return pltpu.make_async_copy(
            x_hbm.at[pl.ds((base + i) * _RB, _RB)], xbuf.at[slot],
            insem.at[slot])

    def dma_out(slot, i):
        return pltpu.make_async_copy(
            obuf.at[slot], o_hbm.at[pl.ds((base + i) * _RB, _RB)],
            outsem.at[slot])

    for k in range(min(_K, nt_per_core)):
        dma_in(k, k).start()

    def body(i, _):
        slot = jax.lax.rem(i, _K)
        dma_in(slot, i).wait()
        xv = xbuf.at[slot]
        xb = xv[...]
        mean = jnp.sum(xb, axis=-1) * jnp.float32(inv_t)
        h = jnp.dot(mean, w1t_ref[...], preferred_element_type=jnp.float32)
        h = jnp.maximum(h + b1_ref[...], 0.0)
        s = jnp.dot(h, w2t_ref[...], preferred_element_type=jnp.float32)
        gate = jax.nn.sigmoid(s + b2_ref[...])

        @pl.when(i >= _K)
        def _():
            dma_out(slot, i - _K).wait()

        ov = obuf.at[slot]
        ov[...] = xb * gate[:, :, None]
        dma_out(slot, i).start()

        @pl.when(i + _K < nt_per_core)
        def _():
            dma_in(slot, i + _K).start()

        return ()

    jax.lax.fori_loop(0, nt_per_core, body, ())

    for k in range(min(_K, nt_per_core)):
        dma_out(k, 0).wait()


def kernel(x, w1, b1, w2, b2):
    B, C, T = x.shape
    H = w1.shape[0]

    w1t = jnp.asarray(w1, jnp.float32).T
    w2t = jnp.asarray(w2, jnp.float32).T
    b1r = jnp.asarray(b1, jnp.float32).reshape(1, H)
    b2r = jnp.asarray(b2, jnp.float32).reshape(1, C)

    return pl.pallas_call(
        functools.partial(_se_kernel, inv_t=1.0 / T,
                          nt_per_core=B // _RB),
        out_shape=jax.ShapeDtypeStruct((B, C, T), x.dtype),
        grid=(1,),
        in_specs=[
            pl.BlockSpec(memory_space=pl.ANY),
            pl.BlockSpec((C, H), lambda b: (0, 0)),
            pl.BlockSpec((1, H), lambda b: (0, 0)),
            pl.BlockSpec((H, C), lambda b: (0, 0)),
            pl.BlockSpec((1, C), lambda b: (0, 0)),
        ],
        out_specs=pl.BlockSpec(memory_space=pl.ANY),
        scratch_shapes=[
            pltpu.VMEM((_K, _RB, C, T), jnp.float32),
            pltpu.VMEM((_K, _RB, C, T), jnp.float32),
            pltpu.SemaphoreType.DMA((_K,)),
            pltpu.SemaphoreType.DMA((_K,)),
        ],
        compiler_params=pltpu.CompilerParams(
            dimension_semantics=("parallel",),
            vmem_limit_bytes=64 * 1024 * 1024,
        ),
    )(x, w1t, b1r, w2t, b2r)


# speedup vs baseline: 1.0061x; 1.0061x over previous
import functools

import jax
import jax.numpy as jnp
from jax.experimental import pallas as pl
from jax.experimental.pallas import tpu as pltpu

_K = 3
_RB = 4


def _se_kernel(x_hbm, w1t_ref, b1_ref, w2t_ref, b2_ref, o_hbm,
               xbuf, obuf, insem, outsem, *, inv_t, nt_per_step):
    step = pl.program_id(0)
    base = step * nt_per_step

    def dma_in(slot, i):
        return pltpu.make_async_copy(
            x_hbm.at[pl.ds((base + i) * _RB, _RB)], xbuf.at[slot],
            insem.at[slot])

    def dma_out(slot, i):
        return pltpu.make_async_copy(
            obuf.at[slot], o_hbm.at[pl.ds((base + i) * _RB, _RB)],
            outsem.at[slot])

    for k in range(min(_K, nt_per_step)):
        dma_in(k, k).start()

    def body(i, _):
        slot = jax.lax.rem(i, _K)
        dma_in(slot, i).wait()
        xv = xbuf.at[slot]
        xb = xv[...]
        mean = jnp.sum(xb, axis=-1) * jnp.float32(inv_t)
        h = jnp.dot(mean, w1t_ref[...], preferred_element_type=jnp.float32)
        h = jnp.maximum(h + b1_ref[...], 0.0)
        s = jnp.dot(h, w2t_ref[...], preferred_element_type=jnp.float32)
        gate = jax.nn.sigmoid(s + b2_ref[...])

        @pl.when(i >= _K)
        def _():
            dma_out(slot, i - _K).wait()

        ov = obuf.at[slot]
        ov[...] = xb * gate[:, :, None]
        dma_out(slot, i).start()

        @pl.when(i + _K < nt_per_step)
        def _():
            dma_in(slot, i + _K).start()

        return ()

    jax.lax.fori_loop(0, nt_per_step, body, ())

    for k in range(min(_K, nt_per_step)):
        dma_out(k, 0).wait()


def kernel(x, w1, b1, w2, b2):
    B, C, T = x.shape
    H = w1.shape[0]

    w1t = jnp.asarray(w1, jnp.float32).T
    w2t = jnp.asarray(w2, jnp.float32).T
    b1r = jnp.asarray(b1, jnp.float32).reshape(1, H)
    b2r = jnp.asarray(b2, jnp.float32).reshape(1, C)

    return pl.pallas_call(
        functools.partial(_se_kernel, inv_t=1.0 / T,
                          nt_per_step=B // (2 * _RB)),
        out_shape=jax.ShapeDtypeStruct((B, C, T), x.dtype),
        grid=(2,),
        in_specs=[
            pl.BlockSpec(memory_space=pl.ANY),
            pl.BlockSpec((C, H), lambda b: (0, 0)),
            pl.BlockSpec((1, H), lambda b: (0, 0)),
            pl.BlockSpec((H, C), lambda b: (0, 0)),
            pl.BlockSpec((1, C), lambda b: (0, 0)),
        ],
        out_specs=pl.BlockSpec(memory_space=pl.ANY),
        scratch_shapes=[
            pltpu.VMEM((_K, _RB, C, T), jnp.float32),
            pltpu.VMEM((_K, _RB, C, T), jnp.float32),
            pltpu.SemaphoreType.DMA((_K,)),
            pltpu.SemaphoreType.DMA((_K,)),
        ],
        compiler_params=pltpu.CompilerParams(
            dimension_semantics=("parallel",),
            vmem_limit_bytes=64 * 1024 * 1024,
        ),
    )(x, w1t, b1r, w2t, b2r)
